# baseline (speedup 1.0000x reference)
import functools

import jax
import jax.numpy as jnp
from jax import lax
from jax.experimental import pallas as pl
from jax.experimental.pallas import tpu as pltpu
from jax.experimental.pallas import tpu_sc as plsc

B, D, H, O = 8192, 2048, 4096, 5
OP = 16
HT = 512
NH = H // HT
RB = 1024
B1 = B // RB
BS = 1024
NBP = B // BS + 1
SB = NBP * BS

NC, NS, L = 2, 16, 16
NW = NC * NS
RPW = B // NW
CH = 32
NCH = RPW // CH

_BIG_NEG = -1e30


def _root_body(x_ref, w1_ref, b1_ref, w2_ref, b2_ref, p0_ref, can_ref, acc_ref):
    h = pl.program_id(1)

    @pl.when(h == 0)
    def _():
        acc_ref[...] = jnp.broadcast_to(b2_ref[0, 0, :], (RB, 2))

    hh = jnp.maximum(
        jnp.dot(x_ref[...], w1_ref[...], preferred_element_type=jnp.float32)
        + b1_ref[0, 0, :], 0.0)
    acc_ref[...] += jnp.dot(hh, w2_ref[...], preferred_element_type=jnp.float32)

    @pl.when(h == NH - 1)
    def _():
        z = acc_ref[...]
        m = jnp.max(z, axis=1, keepdims=True)
        e = jnp.exp(z - m)
        p = e / jnp.sum(e, axis=1, keepdims=True)
        p0_ref[...] = jnp.max(p, axis=1, keepdims=True)
        can_ref[...] = (z[:, 1:2] > z[:, 0:1]).astype(jnp.int32)


def _root(x, w1, b1, w2, b2):
    return pl.pallas_call(
        _root_body,
        grid=(B1, NH),
        in_specs=[
            pl.BlockSpec((RB, D), lambda b, h: (b, 0)),
            pl.BlockSpec((D, HT), lambda b, h: (0, h)),
            pl.BlockSpec((1, 1, HT), lambda b, h: (h, 0, 0)),
            pl.BlockSpec((HT, 2), lambda b, h: (h, 0)),
            pl.BlockSpec((1, 1, 2), lambda b, h: (0, 0, 0)),
        ],
        out_specs=[
            pl.BlockSpec((RB, 1), lambda b, h: (b, 0)),
            pl.BlockSpec((RB, 1), lambda b, h: (b, 0)),
        ],
        out_shape=[
            jax.ShapeDtypeStruct((B, 1), jnp.float32),
            jax.ShapeDtypeStruct((B, 1), jnp.int32),
        ],
        scratch_shapes=[pltpu.VMEM((RB, 2), jnp.float32)],
        compiler_params=pltpu.CompilerParams(
            dimension_semantics=("arbitrary", "arbitrary")),
    )(x, w1, b1.reshape(NH, 1, HT), w2, b2.reshape(1, 1, 2))


def _route_body(can_ref, dest_ref, n0_ref):
    c = can_ref[...]
    m0 = (c == 0).astype(jnp.float32)
    m1 = 1.0 - m0
    r = lax.broadcasted_iota(jnp.int32, (128, 128), 0)
    q = lax.broadcasted_iota(jnp.int32, (128, 128), 1)
    tri = (r <= q).astype(jnp.float32)
    incl0 = jnp.dot(m0, tri, preferred_element_type=jnp.float32)
    incl1 = jnp.dot(m1, tri, preferred_element_type=jnp.float32)
    rs0 = incl0[:, 127:128]
    rs1 = incl1[:, 127:128]
    rr = lax.broadcasted_iota(jnp.int32, (64, 64), 0)
    qq = lax.broadcasted_iota(jnp.int32, (64, 64), 1)
    low = (qq < rr).astype(jnp.float32)
    off0 = jnp.dot(low, rs0, preferred_element_type=jnp.float32)
    off1 = jnp.dot(low, rs1, preferred_element_type=jnp.float32)
    rank0 = off0 + incl0 - 1.0
    rank1 = off1 + incl1 - 1.0
    n0 = off0[63:64, :] + rs0[63:64, :]
    n1start = jnp.ceil(n0 / BS) * BS
    dest = jnp.where(c == 0, rank0, n1start + rank1)
    dest_ref[...] = dest.astype(jnp.int32)
    n0_ref[...] = n0.astype(jnp.int32)


def _route(cancer):
    return pl.pallas_call(
        _route_body,
        out_shape=[
            jax.ShapeDtypeStruct((64, 128), jnp.int32),
            jax.ShapeDtypeStruct((1, 1), jnp.int32),
        ],
    )(cancer.reshape(64, 128))


def _sc_scatter_x(x, p0, dest):
    mesh = plsc.VectorSubcoreMesh(core_axis_name="c", subcore_axis_name="s")

    @functools.partial(
        pl.kernel,
        out_type=(
            jax.ShapeDtypeStruct((SB, D), jnp.float32),
            jax.ShapeDtypeStruct((SB,), jnp.float32),
        ),
        mesh=mesh,
        scratch_types=[
            pltpu.VMEM((CH,), jnp.int32),
            pltpu.VMEM((CH, D), jnp.float32),
            pltpu.VMEM((CH,), jnp.float32),
            pltpu.SemaphoreType.DMA,
            pltpu.SemaphoreType.DMA,
        ],
    )
    def body(x_hbm, p0_hbm, dest_hbm, xs_hbm, p0s_hbm,
             idx_v, row_v, p0_v, s1, s2):
        wid = lax.axis_index("s") * NC + lax.axis_index("c")
        base = wid * RPW
        for c in range(NCH):
            off = base + c * CH
            pltpu.sync_copy(dest_hbm.at[pl.ds(off, CH)], idx_v)
            pltpu.sync_copy(x_hbm.at[pl.ds(off, CH)], row_v)
            pltpu.sync_copy(p0_hbm.at[pl.ds(off, CH)], p0_v)
            cp1 = pltpu.async_copy(row_v, xs_hbm.at[idx_v], s1)
            cp2 = pltpu.async_copy(p0_v, p0s_hbm.at[idx_v], s2)
            cp1.wait()
            cp2.wait()

    return body(x, p0, dest)


def _leaf_body(e_ref, x_ref, p0_ref, w1_ref, b1_ref, w2_ref, b2_ref,
               pso_ref, acc_ref):
    h = pl.program_id(1)

    @pl.when(h == 0)
    def _():
        acc_ref[...] = jnp.broadcast_to(b2_ref[0, 0, :], (BS, OP))

    hh = jnp.maximum(
        jnp.dot(x_ref[...], w1_ref[0], preferred_element_type=jnp.float32)
        + b1_ref[0, 0, 0, :], 0.0)
    acc_ref[...] += jnp.dot(hh, w2_ref[0], preferred_element_type=jnp.float32)

    @pl.when(h == NH - 1)
    def _():
        z = acc_ref[...]
        m = jnp.max(z, axis=1, keepdims=True)
        e = jnp.exp(z - m)
        p = e / jnp.sum(e, axis=1, keepdims=True)
        p0 = p0_ref[...]
        pred = jnp.argmax(z, axis=1).astype(jnp.float32)[:, None]
        pmf = jnp.max(p, axis=1, keepdims=True) * p0
        pf = p * p0
        ci = lax.broadcasted_iota(jnp.int32, (BS, OP), 1)
        combo = jnp.where(ci == 5, pred, jnp.where(ci == 6, pmf, pf))
        pso_ref[...] = jnp.concatenate(
            [combo, jnp.zeros((BS, 128 - OP), jnp.float32)], axis=1)


def _leaf(expert, xs, p0s, w1s, b1s, w2s, b2s):
    return pl.pallas_call(
        _leaf_body,
        grid_spec=pltpu.PrefetchScalarGridSpec(
            num_scalar_prefetch=1,
            grid=(NBP, NH),
            in_specs=[
                pl.BlockSpec((BS, D), lambda p, h, e: (p, 0)),
                pl.BlockSpec((BS, 1), lambda p, h, e: (p, 0)),
                pl.BlockSpec((1, D, HT), lambda p, h, e: (e[p], 0, h)),
                pl.BlockSpec((1, 1, 1, HT), lambda p, h, e: (e[p], h, 0, 0)),
                pl.BlockSpec((1, HT, OP), lambda p, h, e: (e[p], h, 0)),
                pl.BlockSpec((1, 1, OP), lambda p, h, e: (e[p], 0, 0)),
            ],
            out_specs=[
                pl.BlockSpec((BS, 128), lambda p, h, e: (p, 0)),
            ],
            scratch_shapes=[pltpu.VMEM((BS, OP), jnp.float32)],
        ),
        out_shape=[
            jax.ShapeDtypeStruct((SB, 128), jnp.float32),
        ],
        compiler_params=pltpu.CompilerParams(
            dimension_semantics=("arbitrary", "arbitrary")),
    )(expert, xs, p0s, w1s, b1s, w2s, b2s)


def _sc_gather_out(pso_s, dest):
    mesh = plsc.VectorSubcoreMesh(core_axis_name="c", subcore_axis_name="s")

    @functools.partial(
        pl.kernel,
        out_type=jax.ShapeDtypeStruct((B, 128), jnp.float32),
        mesh=mesh,
        scratch_types=[
            pltpu.VMEM((CH,), jnp.int32),
            pltpu.VMEM((CH, 128), jnp.float32),
            pltpu.SemaphoreType.DMA,
        ],
    )
    def body(pso_hbm, dest_hbm, out_hbm, idx_v, f_v, sem):
        wid = lax.axis_index("s") * NC + lax.axis_index("c")
        base = wid * RPW
        for c in range(NCH):
            off = base + c * CH
            pltpu.sync_copy(dest_hbm.at[pl.ds(off, CH)], idx_v)
            pltpu.async_copy(pso_hbm.at[idx_v], f_v, sem).wait()
            pltpu.sync_copy(f_v, out_hbm.at[pl.ds(off, CH)])

    return body(pso_s, dest)


def kernel(x, root_W1, root_b1, root_W2, root_b2,
           leaf1_W1, leaf1_b1, leaf1_W2, leaf1_b2,
           leaf2_W1, leaf2_b1, leaf2_W2, leaf2_b2):
    prob0, cancer = _root(x, root_W1, root_b1, root_W2, root_b2)
    dest2d, n0 = _route(cancer)
    dest = dest2d.reshape(B)
    xs, p0s = _sc_scatter_x(x, prob0.reshape(B), dest)

    n1start = ((n0[0, 0] + BS - 1) // BS) * BS
    expert = (jnp.arange(NBP, dtype=jnp.int32) * BS >= n1start).astype(jnp.int32)

    pad = jnp.full((H, OP - O), 0.0, jnp.float32)
    w1s = jnp.stack([leaf1_W1, leaf2_W1])
    b1s = jnp.stack([leaf1_b1, leaf2_b1]).reshape(2, NH, 1, HT)
    w2s = jnp.stack([
        jnp.concatenate([leaf1_W2, pad], axis=1),
        jnp.concatenate([leaf2_W2, pad], axis=1),
    ])
    bpad = jnp.full((OP - O,), _BIG_NEG, jnp.float32)
    b2s = jnp.stack([
        jnp.concatenate([leaf1_b2, bpad]),
        jnp.concatenate([leaf2_b2, bpad]),
    ]).reshape(2, 1, OP)

    pso_s = _leaf(expert, xs, p0s.reshape(SB, 1), w1s, b1s, w2s, b2s)[0]

    packed = _sc_gather_out(pso_s, dest)
    predictions = packed[:, 5].astype(jnp.int32)
    prob_final = packed[:, 6]
    final_probabilities = packed[:, :O]
    return predictions, prob_final, final_probabilities


# speedup vs baseline: 1.0161x; 1.0161x over previous
import functools

import jax
import jax.numpy as jnp
from jax import lax
from jax.experimental import pallas as pl
from jax.experimental.pallas import tpu as pltpu
from jax.experimental.pallas import tpu_sc as plsc

B, D, H, O = 8192, 2048, 4096, 5
OP = 16
HT = 512
NH = H // HT
RB = 2048
B1 = B // RB
BS = 2048
NBP = B // BS + 1
SB = NBP * BS

NC, NS, L = 2, 16, 16
NW = NC * NS
RPW = B // NW
CH = 32
NCH = RPW // CH

_BIG_NEG = -1e30


def _root_body(x_ref, w1_ref, b1_ref, w2_ref, b2_ref, p0_ref, can_ref, acc_ref):
    h = pl.program_id(1)

    @pl.when(h == 0)
    def _():
        acc_ref[...] = jnp.broadcast_to(b2_ref[0, 0, :], (RB, 2))

    hh = jnp.maximum(
        jnp.dot(x_ref[...], w1_ref[...], preferred_element_type=jnp.float32)
        + b1_ref[0, 0, :], 0.0)
    acc_ref[...] += jnp.dot(hh, w2_ref[...], preferred_element_type=jnp.float32)

    @pl.when(h == NH - 1)
    def _():
        z = acc_ref[...]
        m = jnp.max(z, axis=1, keepdims=True)
        e = jnp.exp(z - m)
        p = e / jnp.sum(e, axis=1, keepdims=True)
        p0_ref[...] = jnp.max(p, axis=1, keepdims=True)
        can_ref[...] = (z[:, 1:2] > z[:, 0:1]).astype(jnp.int32)


def _root(x, w1, b1, w2, b2):
    return pl.pallas_call(
        _root_body,
        grid=(B1, NH),
        in_specs=[
            pl.BlockSpec((RB, D), lambda b, h: (b, 0)),
            pl.BlockSpec((D, HT), lambda b, h: (0, h)),
            pl.BlockSpec((1, 1, HT), lambda b, h: (h, 0, 0)),
            pl.BlockSpec((HT, 2), lambda b, h: (h, 0)),
            pl.BlockSpec((1, 1, 2), lambda b, h: (0, 0, 0)),
        ],
        out_specs=[
            pl.BlockSpec((RB, 1), lambda b, h: (b, 0)),
            pl.BlockSpec((RB, 1), lambda b, h: (b, 0)),
        ],
        out_shape=[
            jax.ShapeDtypeStruct((B, 1), jnp.float32),
            jax.ShapeDtypeStruct((B, 1), jnp.int32),
        ],
        scratch_shapes=[pltpu.VMEM((RB, 2), jnp.float32)],
        compiler_params=pltpu.CompilerParams(
            dimension_semantics=("arbitrary", "arbitrary")),
    )(x, w1, b1.reshape(NH, 1, HT), w2, b2.reshape(1, 1, 2))


def _route_body(can_ref, dest_ref, n0_ref):
    c = can_ref[...]
    m0 = (c == 0).astype(jnp.float32)
    m1 = 1.0 - m0
    r = lax.broadcasted_iota(jnp.int32, (128, 128), 0)
    q = lax.broadcasted_iota(jnp.int32, (128, 128), 1)
    tri = (r <= q).astype(jnp.float32)
    incl0 = jnp.dot(m0, tri, preferred_element_type=jnp.float32)
    incl1 = jnp.dot(m1, tri, preferred_element_type=jnp.float32)
    rs0 = incl0[:, 127:128]
    rs1 = incl1[:, 127:128]
    rr = lax.broadcasted_iota(jnp.int32, (64, 64), 0)
    qq = lax.broadcasted_iota(jnp.int32, (64, 64), 1)
    low = (qq < rr).astype(jnp.float32)
    off0 = jnp.dot(low, rs0, preferred_element_type=jnp.float32)
    off1 = jnp.dot(low, rs1, preferred_element_type=jnp.float32)
    rank0 = off0 + incl0 - 1.0
    rank1 = off1 + incl1 - 1.0
    n0 = off0[63:64, :] + rs0[63:64, :]
    dest = jnp.where(c == 0, rank0, (SB - 1.0) - rank1)
    dest_ref[...] = dest.astype(jnp.int32)
    n0_ref[...] = n0.astype(jnp.int32)


def _route(cancer):
    return pl.pallas_call(
        _route_body,
        out_shape=[
            jax.ShapeDtypeStruct((64, 128), jnp.int32),
            jax.ShapeDtypeStruct((1, 1), jnp.int32),
        ],
    )(cancer.reshape(64, 128))


def _sc_scatter_x(x, p0, dest):
    mesh = plsc.VectorSubcoreMesh(core_axis_name="c", subcore_axis_name="s")

    @functools.partial(
        pl.kernel,
        out_type=(
            jax.ShapeDtypeStruct((SB, D), jnp.float32),
            jax.ShapeDtypeStruct((SB,), jnp.float32),
        ),
        mesh=mesh,
        scratch_types=[
            pltpu.VMEM((CH,), jnp.int32),
            pltpu.VMEM((CH, D), jnp.float32),
            pltpu.VMEM((CH,), jnp.float32),
            pltpu.SemaphoreType.DMA,
            pltpu.SemaphoreType.DMA,
        ],
    )
    def body(x_hbm, p0_hbm, dest_hbm, xs_hbm, p0s_hbm,
             idx_v, row_v, p0_v, s1, s2):
        wid = lax.axis_index("s") * NC + lax.axis_index("c")
        base = wid * RPW
        for c in range(NCH):
            off = base + c * CH
            pltpu.sync_copy(dest_hbm.at[pl.ds(off, CH)], idx_v)
            pltpu.sync_copy(x_hbm.at[pl.ds(off, CH)], row_v)
            pltpu.sync_copy(p0_hbm.at[pl.ds(off, CH)], p0_v)
            cp1 = pltpu.async_copy(row_v, xs_hbm.at[idx_v], s1)
            cp2 = pltpu.async_copy(p0_v, p0s_hbm.at[idx_v], s2)
            cp1.wait()
            cp2.wait()

    return body(x, p0, dest)


def _leaf_body(e_ref, x_ref, p0_ref, w1_ref, b1_ref, w2_ref, b2_ref,
               pso_ref, acc_ref):
    h = pl.program_id(1)

    @pl.when(h == 0)
    def _():
        acc_ref[...] = jnp.broadcast_to(b2_ref[0, 0, :], (BS, OP))

    hh = jnp.maximum(
        jnp.dot(x_ref[...], w1_ref[0], preferred_element_type=jnp.float32)
        + b1_ref[0, 0, 0, :], 0.0)
    acc_ref[...] += jnp.dot(hh, w2_ref[0], preferred_element_type=jnp.float32)

    @pl.when(h == NH - 1)
    def _():
        z = acc_ref[...]
        m = jnp.max(z, axis=1, keepdims=True)
        e = jnp.exp(z - m)
        p = e / jnp.sum(e, axis=1, keepdims=True)
        p0 = p0_ref[...]
        pred = jnp.argmax(z, axis=1).astype(jnp.float32)[:, None]
        pmf = jnp.max(p, axis=1, keepdims=True) * p0
        pf = p * p0
        ci = lax.broadcasted_iota(jnp.int32, (BS, OP), 1)
        combo = jnp.where(ci == 5, pred, jnp.where(ci == 6, pmf, pf))
        pso_ref[...] = jnp.concatenate(
            [combo, jnp.zeros((BS, 128 - OP), jnp.float32)], axis=1)


def _leaf(expert, xs, p0s, w1s, b1s, w2s, b2s):
    return pl.pallas_call(
        _leaf_body,
        grid_spec=pltpu.PrefetchScalarGridSpec(
            num_scalar_prefetch=1,
            grid=(NBP, NH),
            in_specs=[
                pl.BlockSpec((BS, D), lambda p, h, e: (p, 0)),
                pl.BlockSpec((BS, 1), lambda p, h, e: (p, 0)),
                pl.BlockSpec((1, D, HT), lambda p, h, e: (e[p], 0, h)),
                pl.BlockSpec((1, 1, 1, HT), lambda p, h, e: (e[p], h, 0, 0)),
                pl.BlockSpec((1, HT, OP), lambda p, h, e: (e[p], h, 0)),
                pl.BlockSpec((1, 1, OP), lambda p, h, e: (e[p], 0, 0)),
            ],
            out_specs=[
                pl.BlockSpec((BS, 128), lambda p, h, e: (p, 0)),
            ],
            scratch_shapes=[pltpu.VMEM((BS, OP), jnp.float32)],
        ),
        out_shape=[
            jax.ShapeDtypeStruct((SB, 128), jnp.float32),
        ],
        compiler_params=pltpu.CompilerParams(
            dimension_semantics=("arbitrary", "arbitrary")),
    )(expert, xs, p0s, w1s, b1s, w2s, b2s)


def _sc_gather_out(pso_s, dest):
    mesh = plsc.VectorSubcoreMesh(core_axis_name="c", subcore_axis_name="s")

    @functools.partial(
        pl.kernel,
        out_type=jax.ShapeDtypeStruct((B, 128), jnp.float32),
        mesh=mesh,
        scratch_types=[
            pltpu.VMEM((CH,), jnp.int32),
            pltpu.VMEM((CH, 128), jnp.float32),
            pltpu.SemaphoreType.DMA,
        ],
    )
    def body(pso_hbm, dest_hbm, out_hbm, idx_v, f_v, sem):
        wid = lax.axis_index("s") * NC + lax.axis_index("c")
        base = wid * RPW
        for c in range(NCH):
            off = base + c * CH
            pltpu.sync_copy(dest_hbm.at[pl.ds(off, CH)], idx_v)
            pltpu.async_copy(pso_hbm.at[idx_v], f_v, sem).wait()
            pltpu.sync_copy(f_v, out_hbm.at[pl.ds(off, CH)])

    return body(pso_s, dest)


def kernel(x, root_W1, root_b1, root_W2, root_b2,
           leaf1_W1, leaf1_b1, leaf1_W2, leaf1_b2,
           leaf2_W1, leaf2_b1, leaf2_W2, leaf2_b2):
    prob0, cancer = _root(x, root_W1, root_b1, root_W2, root_b2)
    dest2d, n0 = _route(cancer)
    dest = dest2d.reshape(B)
    xs, p0s = _sc_scatter_x(x, prob0.reshape(B), dest)

    expert = (jnp.arange(NBP, dtype=jnp.int32) * BS > n0[0, 0]).astype(jnp.int32)

    pad = jnp.full((H, OP - O), 0.0, jnp.float32)
    w1s = jnp.stack([leaf1_W1, leaf2_W1])
    b1s = jnp.stack([leaf1_b1, leaf2_b1]).reshape(2, NH, 1, HT)
    w2s = jnp.stack([
        jnp.concatenate([leaf1_W2, pad], axis=1),
        jnp.concatenate([leaf2_W2, pad], axis=1),
    ])
    bpad = jnp.full((OP - O,), _BIG_NEG, jnp.float32)
    b2s = jnp.stack([
        jnp.concatenate([leaf1_b2, bpad]),
        jnp.concatenate([leaf2_b2, bpad]),
    ]).reshape(2, 1, OP)

    pso_s = _leaf(expert, xs, p0s.reshape(SB, 1), w1s, b1s, w2s, b2s)[0]

    packed = _sc_gather_out(pso_s, dest)
    predictions = packed[:, 5].astype(jnp.int32)
    prob_final = packed[:, 6]
    final_probabilities = packed[:, :O]
    return predictions, prob_final, final_probabilities
